# Initial kernel scaffold; baseline (speedup 1.0000x reference)
import jax
import jax.numpy as jnp
from jax.experimental import pallas as pl


def kernel(features, coords, matrice1, w_s1, bn_s1_gamma, bn_s1_beta, w_s2, b_s2, bn1_gamma, bn1_beta):
    raise NotImplementedError("write your pallas kernel here")


# speedup vs baseline: 5.1707x; 5.1707x over previous
import functools

import jax
import jax.numpy as jnp
from jax import lax
from jax.experimental import pallas as pl
from jax.experimental.pallas import tpu as pltpu
from jax.experimental.pallas import tpu_sc as plsc

_B, _CIN, _N, _K, _M, _CO = 8, 3, 1024, 20, 8, 64
_R = _B * _N
_E = _R * _K
_NW = 32
_EPW = _E // _NW
_NPW = _R // _NW
_GN = 2
_NG = _NPW // _GN
_NEG = -3.0e38
_EPS = 1e-5


def _knn_body(ct_ref, idx_ref):
    ct = ct_ref[0]
    ctb = ct.astype(jnp.bfloat16)
    g = lax.dot_general(ctb, ctb, (((1,), (1,)), ((), ())),
                        preferred_element_type=jnp.float32)
    sq = ct * ct
    xcol = jnp.sum(sq, axis=1, keepdims=True)
    xrow = lax.dot_general(jnp.ones((1, 16), jnp.float32), sq,
                           (((1,), (1,)), ((), ())),
                           preferred_element_type=jnp.float32,
                           precision=lax.Precision.HIGHEST)
    p0 = 2.0 * g - xcol - xrow
    base = pl.program_id(0) * _N
    iota = lax.broadcasted_iota(jnp.int32, (_N, _N), 1)

    def step(k, p):
        rmax = jnp.max(p, axis=1, keepdims=True)
        msk = p == rmax
        sel = jnp.min(jnp.where(msk, iota, _N), axis=1)
        idx_ref[0, k, :] = sel + base
        return jnp.where(iota == sel[:, None], _NEG, p)

    lax.fori_loop(0, _K, step, p0)


def _knn(coords_t16):
    return pl.pallas_call(
        _knn_body,
        grid=(_B,),
        in_specs=[pl.BlockSpec((1, _N, 16), lambda b: (b, 0, 0))],
        out_specs=pl.BlockSpec((1, _K, _N), lambda b: (b, 0, 0)),
        out_shape=jax.ShapeDtypeStruct((_B, _K, _N), jnp.int32),
    )(coords_t16)


def _feat_body(ft_ref, m1_ref, pt_ref, ctr_ref):
    ft = ft_ref[...]
    m1 = m1_ref[...]
    wsum = m1[0:3] + m1[3:6]
    dn = (((1,), (0,)), ((), ()))
    pt_ref[...] = lax.dot_general(ft, wsum, dn,
                                  preferred_element_type=jnp.float32,
                                  precision=lax.Precision.HIGHEST)
    ctr_ref[...] = lax.dot_general(ft, m1[0:3], dn,
                                   preferred_element_type=jnp.float32,
                                   precision=lax.Precision.HIGHEST)


def _feat(ft, m1):
    rb = 2048
    d = _M * _CO
    return pl.pallas_call(
        _feat_body,
        grid=(_R // rb,),
        in_specs=[pl.BlockSpec((rb, 3), lambda i: (i, 0)),
                  pl.BlockSpec((6, d), lambda i: (0, 0))],
        out_specs=[pl.BlockSpec((rb, d), lambda i: (i, 0)),
                   pl.BlockSpec((rb, d), lambda i: (i, 0))],
        out_shape=[jax.ShapeDtypeStruct((_R, d), jnp.float32),
                   jax.ShapeDtypeStruct((_R, d), jnp.float32)],
    )(ft, m1)


def _sc_mesh():
    return plsc.VectorSubcoreMesh(core_axis_name="c", subcore_axis_name="s")


def _sc_coords_body(tab_hbm, idx_hbm, out_hbm, tab_v, idx_v, out_v):
    wid = lax.axis_index("s") * 2 + lax.axis_index("c")
    base = wid * _EPW
    pltpu.sync_copy(tab_hbm, tab_v)
    pltpu.sync_copy(idx_hbm.at[pl.ds(base, _EPW)], idx_v)

    def step(e, _):
        idxv = idx_v[pl.ds(e, 16)] * 4
        for c in range(3):
            v = plsc.load_gather(tab_v, [idxv + c])
            out_v[c, pl.ds(e, 16)] = v
        return 0

    lax.fori_loop(0, _EPW // 16, lambda i, _: step(i * 16, _), 0, unroll=4)
    for c in range(3):
        pltpu.sync_copy(out_v.at[c], out_hbm.at[c, pl.ds(base, _EPW)])


def _sc_coords(tab4, idx_flat):
    k = functools.partial(
        pl.kernel,
        out_type=jax.ShapeDtypeStruct((4, _E), jnp.float32),
        mesh=_sc_mesh(),
        compiler_params=pltpu.CompilerParams(needs_layout_passes=False),
        scratch_types=[pltpu.VMEM((_R * 4,), jnp.float32),
                       pltpu.VMEM((_EPW,), jnp.int32),
                       pltpu.VMEM((4, _EPW), jnp.float32)],
    )(_sc_coords_body)
    return k(tab4, idx_flat)


def _s1_mats(w1):
    w1t = w1.T
    z = jnp.zeros((1, 16), jnp.float32)
    a4 = jnp.concatenate([w1t[0:3] + w1t[3:6], z], axis=0)
    b4 = jnp.concatenate([-w1t[0:3], z], axis=0)
    return a4, b4


def _s1_of(nbr4, ctr, a4, b4):
    return (lax.dot_general(nbr4, a4, (((0,), (0,)), ((), ())),
                            preferred_element_type=jnp.float32,
                            precision=lax.Precision.HIGHEST)
            + lax.dot_general(ctr, b4, (((1,), (0,)), ((), ())),
                              preferred_element_type=jnp.float32,
                              precision=lax.Precision.HIGHEST))


def _score_stats_body(nbr_ref, ctr_ref, w1_ref, acc_ref):
    a16, b16 = _s1_mats(w1_ref[...])
    s1 = _s1_of(nbr_ref[...], ctr_ref[...], a16, b16)
    us = jnp.sum(s1, axis=0, keepdims=True)
    us2 = jnp.sum(s1 * s1, axis=0, keepdims=True)
    zl = jnp.zeros((1, 112), jnp.float32)
    upd = jnp.concatenate(
        [jnp.concatenate([us, zl], axis=1),
         jnp.concatenate([us2, zl], axis=1),
         jnp.zeros((6, 128), jnp.float32)], axis=0)

    @pl.when(pl.program_id(0) == 0)
    def _():
        acc_ref[...] = upd

    @pl.when(pl.program_id(0) != 0)
    def _():
        acc_ref[...] = acc_ref[...] + upd


def _score_apply_body(nbr_ref, ctr_ref, w1_ref, w2_ref, bs2_ref, g1_ref,
                      b1_ref, acc_ref, sc_ref):
    a16, b16 = _s1_mats(w1_ref[...])
    s1 = _s1_of(nbr_ref[...], ctr_ref[...], a16, b16)
    cnt = jnp.float32(_E)
    mean = acc_ref[0:1, 0:16] / cnt
    var = acc_ref[1:2, 0:16] / cnt - mean * mean
    scale = g1_ref[...] / jnp.sqrt(var + _EPS)
    shift = b1_ref[...] - mean * scale
    s1r = jnp.maximum(s1 * scale + shift, 0.0)
    s2 = lax.dot_general(s1r, w2_ref[...], (((1,), (1,)), ((), ())),
                         preferred_element_type=jnp.float32,
                         precision=lax.Precision.HIGHEST)
    s2 = s2 + bs2_ref[...]
    mx = jnp.max(s2, axis=1, keepdims=True)
    e = jnp.exp(s2 - mx)
    sc_ref[...] = e / jnp.sum(e, axis=1, keepdims=True) + 0.5


def _score(nbrc, ctr_rep, w1, w2, bs2, g1, b1):
    rb = 2560
    grid = (_E // rb,)
    ns = [pl.BlockSpec((4, rb), lambda i: (0, i)),
          pl.BlockSpec((rb, 4), lambda i: (i, 0)),
          pl.BlockSpec((16, 6), lambda i: (0, 0))]
    acc = pl.pallas_call(
        _score_stats_body,
        grid=grid,
        in_specs=ns,
        out_specs=pl.BlockSpec((8, 128), lambda i: (0, 0)),
        out_shape=jax.ShapeDtypeStruct((8, 128), jnp.float32),
    )(nbrc, ctr_rep, w1)
    score = pl.pallas_call(
        _score_apply_body,
        grid=grid,
        in_specs=ns + [pl.BlockSpec((8, 16), lambda i: (0, 0)),
                       pl.BlockSpec((1, 8), lambda i: (0, 0)),
                       pl.BlockSpec((1, 16), lambda i: (0, 0)),
                       pl.BlockSpec((1, 16), lambda i: (0, 0)),
                       pl.BlockSpec((8, 128), lambda i: (0, 0))],
        out_specs=pl.BlockSpec((rb, 8), lambda i: (i, 0)),
        out_shape=jax.ShapeDtypeStruct((_E, 8), jnp.float32),
    )(nbrc, ctr_rep, w1, w2, bs2, g1, b1, acc)
    return score


def _sc_aggr_body(pt_hbm, idx_hbm, sc_hbm, out_hbm,
                  idx_v, sc_v0, sc_v1, rows_v, out_v, sem0, sem1):
    wid = lax.axis_index("s") * 2 + lax.axis_index("c")
    nbase = wid * _NPW
    pltpu.sync_copy(idx_hbm.at[pl.ds(nbase * _K, _NPW * _K)], idx_v)
    sems = (sem0, sem1)
    scs = (sc_v0, sc_v1)
    gsc = _GN * _K * 8

    def start(g, b):
        pltpu.async_copy(
            pt_hbm.at[idx_v.at[pl.ds(g * (_GN * _K), _GN * _K)]],
            rows_v.at[b], sems[b])
        pltpu.async_copy(
            sc_hbm.at[pl.ds(nbase * _K * 8 + g * gsc, gsc)],
            scs[b], sems[b])

    def wait(b):
        pltpu.make_async_copy(
            pt_hbm.at[pl.ds(0, _GN * _K)], rows_v.at[b], sems[b]).wait()
        pltpu.make_async_copy(
            sc_hbm.at[pl.ds(0, gsc)], scs[b], sems[b]).wait()

    for b in range(2):
        start(jnp.int32(b), b)

    def go_body(go, _):
        for b in range(2):
            g = go * 2 + b
            wait(b)
            for nl in range(_GN):
                def kstep(k, accs):
                    r = nl * _K + k
                    accs = list(accs)
                    for m in range(_M):
                        sv = plsc.load_gather(
                            scs[b],
                            [jnp.full((16,), nl * (_K * 8) + k * 8 + m,
                                      jnp.int32)])
                        for c in range(4):
                            accs[c] = accs[c] + sv * rows_v[
                                b, r, pl.ds(m * _CO + c * 16, 16)]
                    return tuple(accs)

                accs = tuple(jnp.zeros((16,), jnp.float32)
                             for _ in range(4))
                accs = lax.fori_loop(0, _K, kstep, accs)
                nrow = g * _GN + nl
                for c in range(4):
                    out_v[nrow, pl.ds(c * 16, 16)] = accs[c]

            @pl.when(g + 2 < _NG)
            def _():
                start(g + 2, b)
        return 0

    lax.fori_loop(0, _NG // 2, go_body, 0)
    pltpu.sync_copy(out_v, out_hbm.at[pl.ds(nbase, _NPW)])


def _sc_aggr(pt, idx_flat, score_flat):
    k = functools.partial(
        pl.kernel,
        out_type=jax.ShapeDtypeStruct((_R, _CO), jnp.float32),
        mesh=_sc_mesh(),
        compiler_params=pltpu.CompilerParams(needs_layout_passes=False),
        scratch_types=[pltpu.VMEM((_NPW * _K,), jnp.int32),
                       pltpu.VMEM((_GN * _K * 8,), jnp.float32),
                       pltpu.VMEM((_GN * _K * 8,), jnp.float32),
                       pltpu.VMEM((2, _GN * _K, _M * _CO), jnp.float32),
                       pltpu.VMEM((_NPW, _CO), jnp.float32),
                       pltpu.SemaphoreType.DMA,
                       pltpu.SemaphoreType.DMA],
    )(_sc_aggr_body)
    return k(pt, idx_flat, score_flat)


def _fin_stats_body(sc2_ref, ctrf_ref, o1_ref, d_ref, acc_ref):
    sc2 = sc2_ref[...]
    ctrf = ctrf_ref[...]
    ss = sc2[:, 0:8]
    for k in range(1, _K):
        ss = ss + sc2[:, k * 8:(k + 1) * 8]
    out2 = ss[:, 0:1] * ctrf[:, 0:_CO]
    for m in range(1, _M):
        out2 = out2 + ss[:, m:m + 1] * ctrf[:, m * _CO:(m + 1) * _CO]
    dd = o1_ref[...] - out2
    d_ref[...] = dd
    us = jnp.sum(dd, axis=0, keepdims=True)
    us2 = jnp.sum(dd * dd, axis=0, keepdims=True)
    zl = jnp.zeros((1, 64), jnp.float32)
    upd = jnp.concatenate(
        [jnp.concatenate([us, zl], axis=1),
         jnp.concatenate([us2, zl], axis=1),
         jnp.zeros((6, 128), jnp.float32)], axis=0)

    @pl.when(pl.program_id(0) == 0)
    def _():
        acc_ref[...] = upd

    @pl.when(pl.program_id(0) != 0)
    def _():
        acc_ref[...] = acc_ref[...] + upd


def _fin_apply_body(d_ref, acc_ref, g_ref, b_ref, out_ref):
    cnt = jnp.float32(_R)
    mean = acc_ref[0:1, 0:_CO] / cnt
    var = acc_ref[1:2, 0:_CO] / cnt - mean * mean
    scale = g_ref[...] / jnp.sqrt(var + _EPS)
    shift = b_ref[...] - mean * scale
    y = jnp.maximum(d_ref[...] * scale + shift, 0.0)
    out_ref[0] = y.T


def _finalize(score2d, ctrf, out1, bn1_g, bn1_b):
    rb = 512
    d_arr, acc = pl.pallas_call(
        _fin_stats_body,
        grid=(_R // rb,),
        in_specs=[pl.BlockSpec((rb, _K * 8), lambda i: (i, 0)),
                  pl.BlockSpec((rb, _M * _CO), lambda i: (i, 0)),
                  pl.BlockSpec((rb, _CO), lambda i: (i, 0))],
        out_specs=[pl.BlockSpec((rb, _CO), lambda i: (i, 0)),
                   pl.BlockSpec((8, 128), lambda i: (0, 0))],
        out_shape=[jax.ShapeDtypeStruct((_R, _CO), jnp.float32),
                   jax.ShapeDtypeStruct((8, 128), jnp.float32)],
    )(score2d, ctrf, out1)
    nb = _N // rb
    out = pl.pallas_call(
        _fin_apply_body,
        grid=(_B, nb),
        in_specs=[pl.BlockSpec((rb, _CO), lambda b, j: (b * nb + j, 0)),
                  pl.BlockSpec((8, 128), lambda b, j: (0, 0)),
                  pl.BlockSpec((1, _CO), lambda b, j: (0, 0)),
                  pl.BlockSpec((1, _CO), lambda b, j: (0, 0))],
        out_specs=pl.BlockSpec((1, _CO, rb), lambda b, j: (b, 0, j)),
        out_shape=jax.ShapeDtypeStruct((_B, _CO, _N), jnp.float32),
    )(d_arr, acc, bn1_g, bn1_b)
    return out


def kernel(features, coords, matrice1, w_s1, bn_s1_gamma, bn_s1_beta,
           w_s2, b_s2, bn1_gamma, bn1_beta):
    coords_t = jnp.transpose(coords, (0, 2, 1))
    ct16 = jnp.pad(coords_t, ((0, 0), (0, 0), (0, 13)))

    idx_bkn = _knn(ct16)
    idx_flat = jnp.transpose(idx_bkn, (0, 2, 1)).reshape(_E)

    ft = jnp.transpose(features, (0, 2, 1)).reshape(_R, _CIN)
    point, ctrf = _feat(ft, matrice1)

    ctab4 = jnp.pad(coords_t, ((0, 0), (0, 0), (0, 1))).reshape(_R, 4)
    nbrc = _sc_coords(ctab4.reshape(_R * 4), idx_flat)
    ctr_rep = jnp.broadcast_to(
        ctab4[:, None, :], (_R, _K, 4)).reshape(_E, 4)

    score = _score(nbrc, ctr_rep, w_s1, w_s2,
                   b_s2.reshape(1, 8), bn_s1_gamma.reshape(1, 16),
                   bn_s1_beta.reshape(1, 16))

    out1 = _sc_aggr(point, idx_flat, score.reshape(_E * 8))

    out = _finalize(score.reshape(_R, _K * 8), ctrf, out1,
                    bn1_gamma.reshape(1, _CO), bn1_beta.reshape(1, _CO))
    return out


# speedup vs baseline: 7.5580x; 1.4617x over previous
import functools

import jax
import jax.numpy as jnp
from jax import lax
from jax.experimental import pallas as pl
from jax.experimental.pallas import tpu as pltpu
from jax.experimental.pallas import tpu_sc as plsc

_B, _CIN, _N, _K, _M, _CO = 8, 3, 1024, 20, 8, 64
_R = _B * _N
_E = _R * _K
_NW = 32
_EPW = _E // _NW
_NPW = _R // _NW
_GN = 2
_NG = _NPW // _GN
_NEG = -3.0e38
_EPS = 1e-5


def _knn_body(ct_ref, idx_ref):
    ct = ct_ref[0]
    ctb = ct.astype(jnp.bfloat16)
    g = lax.dot_general(ctb, ctb, (((1,), (1,)), ((), ())),
                        preferred_element_type=jnp.float32)
    sq = ct * ct
    xcol = jnp.sum(sq, axis=1, keepdims=True)
    xrow = lax.dot_general(jnp.ones((1, 16), jnp.float32), sq,
                           (((1,), (1,)), ((), ())),
                           preferred_element_type=jnp.float32,
                           precision=lax.Precision.HIGHEST)
    p0 = 2.0 * g - xcol - xrow
    base = pl.program_id(0) * _N
    iota = lax.broadcasted_iota(jnp.int32, (_N, _N), 1)

    def step(k, p):
        rmax = jnp.max(p, axis=1, keepdims=True)
        msk = p == rmax
        sel = jnp.min(jnp.where(msk, iota, _N), axis=1)
        idx_ref[0, k, :] = sel + base
        return jnp.where(iota == sel[:, None], _NEG, p)

    lax.fori_loop(0, _K, step, p0)


def _knn(coords_t16):
    return pl.pallas_call(
        _knn_body,
        grid=(_B,),
        in_specs=[pl.BlockSpec((1, _N, 16), lambda b: (b, 0, 0))],
        out_specs=pl.BlockSpec((1, _K, _N), lambda b: (b, 0, 0)),
        out_shape=jax.ShapeDtypeStruct((_B, _K, _N), jnp.int32),
    )(coords_t16)


def _feat_body(ft_ref, m1_ref, pt_ref, ctr_ref):
    ft = ft_ref[...]
    m1 = m1_ref[...]
    wsum = m1[0:3] + m1[3:6]
    dn = (((1,), (0,)), ((), ()))
    pt_ref[...] = lax.dot_general(ft, wsum, dn,
                                  preferred_element_type=jnp.float32,
                                  precision=lax.Precision.HIGHEST)
    ctr_ref[...] = lax.dot_general(ft, m1[0:3], dn,
                                   preferred_element_type=jnp.float32,
                                   precision=lax.Precision.HIGHEST)


def _feat(ft, m1):
    rb = 2048
    d = _M * _CO
    return pl.pallas_call(
        _feat_body,
        grid=(_R // rb,),
        in_specs=[pl.BlockSpec((rb, 3), lambda i: (i, 0)),
                  pl.BlockSpec((6, d), lambda i: (0, 0))],
        out_specs=[pl.BlockSpec((rb, d), lambda i: (i, 0)),
                   pl.BlockSpec((rb, d), lambda i: (i, 0))],
        out_shape=[jax.ShapeDtypeStruct((_R, d), jnp.float32),
                   jax.ShapeDtypeStruct((_R, d), jnp.float32)],
    )(ft, m1)


def _sc_mesh():
    return plsc.VectorSubcoreMesh(core_axis_name="c", subcore_axis_name="s")


def _sc_coords_body(tab_hbm, idx_hbm, out_hbm, tab_v, idx_v, out_v):
    wid = lax.axis_index("s") * 2 + lax.axis_index("c")
    base = wid * _EPW
    pltpu.sync_copy(tab_hbm, tab_v)
    pltpu.sync_copy(idx_hbm.at[pl.ds(base, _EPW)], idx_v)

    def step(e, _):
        idxv = idx_v[pl.ds(e, 16)] * 4
        for c in range(3):
            v = plsc.load_gather(tab_v, [idxv + c])
            out_v[c, pl.ds(e, 16)] = v
        return 0

    lax.fori_loop(0, _EPW // 16, lambda i, _: step(i * 16, _), 0, unroll=4)
    for c in range(3):
        pltpu.sync_copy(out_v.at[c], out_hbm.at[c, pl.ds(base, _EPW)])


def _sc_coords(tab4, idx_flat):
    k = functools.partial(
        pl.kernel,
        out_type=jax.ShapeDtypeStruct((4, _E), jnp.float32),
        mesh=_sc_mesh(),
        compiler_params=pltpu.CompilerParams(needs_layout_passes=False),
        scratch_types=[pltpu.VMEM((_R * 4,), jnp.float32),
                       pltpu.VMEM((_EPW,), jnp.int32),
                       pltpu.VMEM((4, _EPW), jnp.float32)],
    )(_sc_coords_body)
    return k(tab4, idx_flat)


def _s1_mats(w1):
    w1t = w1.T
    z = jnp.zeros((1, 16), jnp.float32)
    a4 = jnp.concatenate([w1t[0:3] + w1t[3:6], z], axis=0)
    b4 = jnp.concatenate([-w1t[0:3], z], axis=0)
    return a4, b4


def _s1_of(nbr4, ctr4, a4, b4):
    return (lax.dot_general(a4, nbr4, (((0,), (0,)), ((), ())),
                            preferred_element_type=jnp.float32,
                            precision=lax.Precision.HIGHEST)
            + lax.dot_general(b4, ctr4, (((0,), (0,)), ((), ())),
                              preferred_element_type=jnp.float32,
                              precision=lax.Precision.HIGHEST))


def _score_stats_body(nbr_ref, ctr_ref, w1_ref, acc_ref):
    a4, b4 = _s1_mats(w1_ref[...])
    s1 = _s1_of(nbr_ref[...], ctr_ref[...], a4, b4)
    us = jnp.sum(s1, axis=1, keepdims=True)
    us2 = jnp.sum(s1 * s1, axis=1, keepdims=True)
    upd = jnp.concatenate(
        [us, us2, jnp.zeros((16, 126), jnp.float32)], axis=1)

    @pl.when(pl.program_id(0) == 0)
    def _():
        acc_ref[...] = upd

    @pl.when(pl.program_id(0) != 0)
    def _():
        acc_ref[...] = acc_ref[...] + upd


def _score_apply_body(nbr_ref, ctr_ref, w1_ref, w2_ref, bs2_ref, g1_ref,
                      b1_ref, acc_ref, sc_ref):
    a4, b4 = _s1_mats(w1_ref[...])
    s1 = _s1_of(nbr_ref[...], ctr_ref[...], a4, b4)
    cnt = jnp.float32(_E)
    mean = acc_ref[:, 0:1] / cnt
    var = acc_ref[:, 1:2] / cnt - mean * mean
    scale = g1_ref[...] / jnp.sqrt(var + _EPS)
    shift = b1_ref[...] - mean * scale
    s1r = jnp.maximum(s1 * scale + shift, 0.0)
    s2 = lax.dot_general(w2_ref[...], s1r, (((1,), (0,)), ((), ())),
                         preferred_element_type=jnp.float32,
                         precision=lax.Precision.HIGHEST)
    s2 = s2 + bs2_ref[...]
    mx = jnp.max(s2, axis=0, keepdims=True)
    e = jnp.exp(s2 - mx)
    sm = e / jnp.sum(e, axis=0, keepdims=True) + 0.5
    sc_ref[...] = sm.T


def _score(nbrc, ctr4p, w1, w2, bs2, g1, b1):
    rb = 2560
    grid = (_E // rb,)
    ns = [pl.BlockSpec((4, rb), lambda i: (0, i)),
          pl.BlockSpec((4, rb), lambda i: (0, i)),
          pl.BlockSpec((16, 6), lambda i: (0, 0))]
    acc = pl.pallas_call(
        _score_stats_body,
        grid=grid,
        in_specs=ns,
        out_specs=pl.BlockSpec((16, 128), lambda i: (0, 0)),
        out_shape=jax.ShapeDtypeStruct((16, 128), jnp.float32),
    )(nbrc, ctr4p, w1)
    score = pl.pallas_call(
        _score_apply_body,
        grid=grid,
        in_specs=ns + [pl.BlockSpec((8, 16), lambda i: (0, 0)),
                       pl.BlockSpec((8, 1), lambda i: (0, 0)),
                       pl.BlockSpec((16, 1), lambda i: (0, 0)),
                       pl.BlockSpec((16, 1), lambda i: (0, 0)),
                       pl.BlockSpec((16, 128), lambda i: (0, 0))],
        out_specs=pl.BlockSpec((rb, 8), lambda i: (i, 0)),
        out_shape=jax.ShapeDtypeStruct((_E, 8), jnp.float32),
    )(nbrc, ctr4p, w1, w2, bs2, g1, b1, acc)
    return score


def _sc_aggr_body(pt_hbm, idx_hbm, sc_hbm, out_hbm,
                  idx_v, sc_v0, sc_v1, rows_v, out_v, sem0, sem1):
    wid = lax.axis_index("s") * 2 + lax.axis_index("c")
    nbase = wid * _NPW
    pltpu.sync_copy(idx_hbm.at[pl.ds(nbase * _K, _NPW * _K)], idx_v)
    sems = (sem0, sem1)
    scs = (sc_v0, sc_v1)
    gsc = _GN * _K * 8

    def start(g, b):
        pltpu.async_copy(
            pt_hbm.at[idx_v.at[pl.ds(g * (_GN * _K), _GN * _K)]],
            rows_v.at[b], sems[b])
        pltpu.async_copy(
            sc_hbm.at[pl.ds(nbase * _K * 8 + g * gsc, gsc)],
            scs[b], sems[b])

    def wait(b):
        pltpu.make_async_copy(
            pt_hbm.at[pl.ds(0, _GN * _K)], rows_v.at[b], sems[b]).wait()
        pltpu.make_async_copy(
            sc_hbm.at[pl.ds(0, gsc)], scs[b], sems[b]).wait()

    for b in range(2):
        start(jnp.int32(b), b)

    def go_body(go, _):
        for b in range(2):
            g = go * 2 + b
            wait(b)
            for nl in range(_GN):
                def kstep(k, accs):
                    r = nl * _K + k
                    accs = list(accs)
                    for m in range(_M):
                        sv = plsc.load_gather(
                            scs[b],
                            [jnp.full((16,), nl * (_K * 8) + k * 8 + m,
                                      jnp.int32)])
                        for c in range(4):
                            accs[c] = accs[c] + sv * rows_v[
                                b, r, pl.ds(m * _CO + c * 16, 16)]
                    return tuple(accs)

                accs = tuple(jnp.zeros((16,), jnp.float32)
                             for _ in range(4))
                accs = lax.fori_loop(0, _K, kstep, accs)
                nrow = g * _GN + nl
                for c in range(4):
                    out_v[nrow, pl.ds(c * 16, 16)] = accs[c]

            @pl.when(g + 2 < _NG)
            def _():
                start(g + 2, b)
        return 0

    lax.fori_loop(0, _NG // 2, go_body, 0)
    pltpu.sync_copy(out_v, out_hbm.at[pl.ds(nbase, _NPW)])


def _sc_aggr(pt, idx_flat, score_flat):
    k = functools.partial(
        pl.kernel,
        out_type=jax.ShapeDtypeStruct((_R, _CO), jnp.float32),
        mesh=_sc_mesh(),
        compiler_params=pltpu.CompilerParams(needs_layout_passes=False),
        scratch_types=[pltpu.VMEM((_NPW * _K,), jnp.int32),
                       pltpu.VMEM((_GN * _K * 8,), jnp.float32),
                       pltpu.VMEM((_GN * _K * 8,), jnp.float32),
                       pltpu.VMEM((2, _GN * _K, _M * _CO), jnp.float32),
                       pltpu.VMEM((_NPW, _CO), jnp.float32),
                       pltpu.SemaphoreType.DMA,
                       pltpu.SemaphoreType.DMA],
    )(_sc_aggr_body)
    return k(pt, idx_flat, score_flat)


def _fin_stats_body(sc2_ref, ctrf_ref, o1_ref, d_ref, acc_ref):
    sc2 = sc2_ref[...]
    ctrf = ctrf_ref[...]
    ss = sc2[:, 0:8]
    for k in range(1, _K):
        ss = ss + sc2[:, k * 8:(k + 1) * 8]
    out2 = ss[:, 0:1] * ctrf[:, 0:_CO]
    for m in range(1, _M):
        out2 = out2 + ss[:, m:m + 1] * ctrf[:, m * _CO:(m + 1) * _CO]
    dd = o1_ref[...] - out2
    d_ref[...] = dd
    us = jnp.sum(dd, axis=0, keepdims=True)
    us2 = jnp.sum(dd * dd, axis=0, keepdims=True)
    zl = jnp.zeros((1, 64), jnp.float32)
    upd = jnp.concatenate(
        [jnp.concatenate([us, zl], axis=1),
         jnp.concatenate([us2, zl], axis=1),
         jnp.zeros((6, 128), jnp.float32)], axis=0)

    @pl.when(pl.program_id(0) == 0)
    def _():
        acc_ref[...] = upd

    @pl.when(pl.program_id(0) != 0)
    def _():
        acc_ref[...] = acc_ref[...] + upd


def _fin_apply_body(d_ref, acc_ref, g_ref, b_ref, out_ref):
    cnt = jnp.float32(_R)
    mean = acc_ref[0:1, 0:_CO] / cnt
    var = acc_ref[1:2, 0:_CO] / cnt - mean * mean
    scale = g_ref[...] / jnp.sqrt(var + _EPS)
    shift = b_ref[...] - mean * scale
    y = jnp.maximum(d_ref[...] * scale + shift, 0.0)
    out_ref[0] = y.T


def _finalize(score2d, ctrf, out1, bn1_g, bn1_b):
    rb = 512
    d_arr, acc = pl.pallas_call(
        _fin_stats_body,
        grid=(_R // rb,),
        in_specs=[pl.BlockSpec((rb, _K * 8), lambda i: (i, 0)),
                  pl.BlockSpec((rb, _M * _CO), lambda i: (i, 0)),
                  pl.BlockSpec((rb, _CO), lambda i: (i, 0))],
        out_specs=[pl.BlockSpec((rb, _CO), lambda i: (i, 0)),
                   pl.BlockSpec((8, 128), lambda i: (0, 0))],
        out_shape=[jax.ShapeDtypeStruct((_R, _CO), jnp.float32),
                   jax.ShapeDtypeStruct((8, 128), jnp.float32)],
    )(score2d, ctrf, out1)
    nb = _N // rb
    out = pl.pallas_call(
        _fin_apply_body,
        grid=(_B, nb),
        in_specs=[pl.BlockSpec((rb, _CO), lambda b, j: (b * nb + j, 0)),
                  pl.BlockSpec((8, 128), lambda b, j: (0, 0)),
                  pl.BlockSpec((1, _CO), lambda b, j: (0, 0)),
                  pl.BlockSpec((1, _CO), lambda b, j: (0, 0))],
        out_specs=pl.BlockSpec((1, _CO, rb), lambda b, j: (b, 0, j)),
        out_shape=jax.ShapeDtypeStruct((_B, _CO, _N), jnp.float32),
    )(d_arr, acc, bn1_g, bn1_b)
    return out


def kernel(features, coords, matrice1, w_s1, bn_s1_gamma, bn_s1_beta,
           w_s2, b_s2, bn1_gamma, bn1_beta):
    coords_t = jnp.transpose(coords, (0, 2, 1))
    ct16 = jnp.pad(coords_t, ((0, 0), (0, 0), (0, 13)))

    idx_bkn = _knn(ct16)
    idx_flat = jnp.transpose(idx_bkn, (0, 2, 1)).reshape(_E)

    ft = jnp.transpose(features, (0, 2, 1)).reshape(_R, _CIN)
    point, ctrf = _feat(ft, matrice1)

    ctab4 = jnp.pad(coords_t, ((0, 0), (0, 0), (0, 1))).reshape(_R, 4)
    nbrc = _sc_coords(ctab4.reshape(_R * 4), idx_flat)
    ctr4p = jnp.broadcast_to(
        ctab4.T[:, :, None], (4, _R, _K)).reshape(4, _E)

    score = _score(nbrc, ctr4p, w_s1, w_s2,
                   b_s2.reshape(8, 1), bn_s1_gamma.reshape(16, 1),
                   bn_s1_beta.reshape(16, 1))

    out1 = _sc_aggr(point, idx_flat, score.reshape(_E * 8))

    out = _finalize(score.reshape(_R, _K * 8), ctrf, out1,
                    bn1_gamma.reshape(1, _CO), bn1_beta.reshape(1, _CO))
    return out
